# baseline (speedup 1.0000x reference)
import functools

import jax
import jax.numpy as jnp
from jax import lax
from jax.experimental import pallas as pl
from jax.experimental.pallas import tpu as pltpu


def _slot_rnn_fwd(tok_sm, table_hbm, whh_hbm, wih0_hbm, wih1_hbm, wlin_hbm,
                  bias0i_ref, ghb_ref, bias1i_ref, blin_ref,
                  o_ref, slab_ref, emb_ref, gi_ref, h1_ref,
                  wih0_ref, w0_ref, w1_ref, wih1_ref, wlin_ref,
                  gsem, wsem):
    i = pl.program_id(0)
    Bs, T, C = o_ref.shape
    E = emb_ref.shape[1]
    H3 = wih0_ref.shape[1]
    H = H3 // 3
    M = Bs * T
    npiece = 2
    piece_sz = M // npiece
    f32 = jnp.float32

    toks = []
    for t in range(T):
        for b in range(Bs):
            tok = tok_sm[i * Bs + b, t]
            toks.append(tok)
            base = pl.multiple_of((tok >> 3) << 3, 8)
            slot = t * Bs + b
            pltpu.make_async_copy(
                table_hbm.at[pl.ds(base, 8), :],
                slab_ref.at[slot], gsem.at[slot // piece_sz]).start()
        if t == T // npiece - 1:
            pltpu.make_async_copy(wih0_hbm, wih0_ref, wsem.at[0]).start()

    pltpu.make_async_copy(
        whh_hbm.at[pl.ds(0, H), pl.ds(0, H3)], w0_ref, wsem.at[1]).start()
    pltpu.make_async_copy(
        whh_hbm.at[pl.ds(H, H), pl.ds(H3, H3)], w1_ref, wsem.at[2]).start()
    pltpu.make_async_copy(wih1_hbm, wih1_ref, wsem.at[3]).start()
    pltpu.make_async_copy(wlin_hbm, wlin_ref, wsem.at[4]).start()

    b0i = bias0i_ref[...]
    iota8 = lax.broadcasted_iota(jnp.int32, (8, E), 0)
    for piece in range(npiece):
        rows = pl.ds(piece * piece_sz, piece_sz)
        pltpu.make_async_copy(
            slab_ref.at[rows], slab_ref.at[rows], gsem.at[piece]).wait()
        for slot in range(piece * piece_sz, (piece + 1) * piece_sz):
            sel = (iota8 == (toks[slot] & 7)).astype(f32)
            emb_ref[pl.ds(slot, 1), :] = jnp.sum(
                slab_ref[slot] * sel, axis=0, keepdims=True)
        if piece == 0:
            pltpu.make_async_copy(wih0_ref, wih0_ref, wsem.at[0]).wait()
            wih0 = wih0_ref[...]
        else:
            gi_ref[...] = (jnp.dot(emb_ref[...], wih0,
                                   preferred_element_type=f32) + b0i)

    pltpu.make_async_copy(w0_ref, w0_ref, wsem.at[1]).wait()
    pltpu.make_async_copy(wih1_ref, wih1_ref, wsem.at[3]).wait()
    pltpu.make_async_copy(w1_ref, w1_ref, wsem.at[2]).wait()
    pltpu.make_async_copy(wlin_ref, wlin_ref, wsem.at[4]).wait()
    w0 = w0_ref[...]
    w1 = w1_ref[...]
    wih1 = wih1_ref[...]
    wlin = wlin_ref[...]
    blin = blin_ref[...]
    gb0 = ghb_ref[:, 0:H3]
    gb1 = ghb_ref[:, H3:2 * H3]
    b1i = bias1i_ref[...]

    def cell(gi, gh, h):
        rz = jax.nn.sigmoid(gi[:, :2 * H] + gh[:, :2 * H])
        n = jnp.tanh(gi[:, 2 * H:] + rz[:, :H] * gh[:, 2 * H:])
        if h is None:
            return n - rz[:, H:] * n
        return n + rz[:, H:] * (h - n)

    gb0_b = jnp.broadcast_to(gb0, (Bs, H3))
    gb1_b = jnp.broadcast_to(gb1, (Bs, H3))
    h0 = cell(gi_ref[pl.ds(0, Bs), :], gb0_b, None)
    gi1 = [jnp.dot(h0, wih1, preferred_element_type=f32) + b1i]
    h1 = None
    rows = []
    for t in range(1, T):
        gh0 = jnp.dot(h0, w0, preferred_element_type=f32) + gb0
        if t == 1:
            gh1 = gb1_b
        else:
            gh1 = jnp.dot(h1, w1, preferred_element_type=f32) + gb1
        h1 = cell(gi1[t - 1], gh1, h1)
        rows.append(jnp.dot(h1, wlin, preferred_element_type=f32) + blin)
        h0 = cell(gi_ref[pl.ds(t * Bs, Bs), :], gh0, h0)
        gi1.append(jnp.dot(h0, wih1, preferred_element_type=f32) + b1i)
    gh1 = jnp.dot(h1, w1, preferred_element_type=f32) + gb1
    h1 = cell(gi1[T - 1], gh1, h1)
    rows.append(jnp.dot(h1, wlin, preferred_element_type=f32) + blin)

    m = functools.reduce(jnp.maximum, rows)
    tot = functools.reduce(lambda a, b: a + b,
                           [jnp.exp(r - m) for r in rows])
    lse = m + jnp.log(tot)
    for t in range(T):
        o_ref[:, pl.ds(t, 1), :] = (rows[t] - lse)[:, None, :]


def kernel(tokens, table, wih0, bias0i, whh_blk, gh_bias, wih1, bias1i,
           w_lin, b_lin):
    B, T = tokens.shape
    V, E = table.shape
    C = w_lin.shape[1]
    H3 = wih0.shape[1]
    H = H3 // 3
    G = 1
    Bs = B // G

    def cs(arr):
        nd = arr.ndim
        return pl.BlockSpec(arr.shape, lambda i, tok, _nd=nd: (0,) * _nd)

    hbm = pl.BlockSpec(memory_space=pl.ANY)

    grid_spec = pltpu.PrefetchScalarGridSpec(
        num_scalar_prefetch=1,
        grid=(G,),
        in_specs=[
            hbm,
            hbm,
            hbm,
            hbm,
            hbm,
            cs(bias0i), cs(gh_bias), cs(bias1i), cs(b_lin),
        ],
        out_specs=pl.BlockSpec((Bs, T, C), lambda i, tok: (i, 0, 0)),
        scratch_shapes=[
            pltpu.VMEM((Bs * T, 8, E), jnp.float32),
            pltpu.VMEM((Bs * T, E), jnp.float32),
            pltpu.VMEM((Bs * T, H3), jnp.float32),
            pltpu.VMEM((Bs * T, H), jnp.float32),
            pltpu.VMEM((E, H3), jnp.float32),
            pltpu.VMEM((H, H3), jnp.float32),
            pltpu.VMEM((H, H3), jnp.float32),
            pltpu.VMEM((H, H3), jnp.float32),
            pltpu.VMEM((H, C), jnp.float32),
            pltpu.SemaphoreType.DMA((4,)),
            pltpu.SemaphoreType.DMA((5,)),
        ],
    )
    return pl.pallas_call(
        _slot_rnn_fwd,
        out_shape=jax.ShapeDtypeStruct((B, T, C), jnp.float32),
        grid_spec=grid_spec,
        compiler_params=pltpu.CompilerParams(
            dimension_semantics=("arbitrary",)),
    )(tokens, table, whh_blk, wih0, wih1, w_lin,
      bias0i, gh_bias, bias1i, b_lin)


# speedup vs baseline: 1.1345x; 1.1345x over previous
import functools

import jax
import jax.numpy as jnp
from jax import lax
from jax.experimental import pallas as pl
from jax.experimental.pallas import tpu as pltpu


def _slot_rnn_fwd(tok_sm, table_hbm, whh_hbm, wih0_hbm, wih1_hbm, wlin_hbm,
                  bias0i_ref, ghb_ref, bias1i_ref, blin_ref,
                  o_ref, slab_ref, emb_ref, gi_ref, h1_ref,
                  wih0_ref, w0_ref, w1_ref, wih1_ref, wlin_ref,
                  gsem, wsem):
    i = pl.program_id(0)
    Bs, T, C = o_ref.shape
    E = emb_ref.shape[1]
    H3 = wih0_ref.shape[1]
    H = H3 // 3
    M = Bs * T
    npiece = 2
    piece_sz = M // npiece
    f32 = jnp.float32

    toks = []
    for t in range(T):
        for b in range(Bs):
            tok = tok_sm[i * Bs + b, t]
            toks.append(tok)
            base = pl.multiple_of((tok >> 3) << 3, 8)
            slot = t * Bs + b
            pltpu.make_async_copy(
                table_hbm.at[pl.ds(base, 8), :],
                slab_ref.at[slot], gsem.at[slot // piece_sz]).start()
        if t == T // npiece - 1:
            pltpu.make_async_copy(wih0_hbm, wih0_ref, wsem.at[0]).start()

    pltpu.make_async_copy(
        whh_hbm.at[pl.ds(0, H), pl.ds(0, H3)], w0_ref, wsem.at[1]).start()
    pltpu.make_async_copy(
        whh_hbm.at[pl.ds(H, H), pl.ds(H3, H3)], w1_ref, wsem.at[2]).start()
    pltpu.make_async_copy(wih1_hbm, wih1_ref, wsem.at[3]).start()
    pltpu.make_async_copy(wlin_hbm, wlin_ref, wsem.at[4]).start()

    b0i = bias0i_ref[...]
    iota8 = lax.broadcasted_iota(jnp.int32, (8, E), 0)
    for piece in range(npiece):
        rows = pl.ds(piece * piece_sz, piece_sz)
        pltpu.make_async_copy(
            slab_ref.at[rows], slab_ref.at[rows], gsem.at[piece]).wait()
        for slot in range(piece * piece_sz, (piece + 1) * piece_sz):
            sel = (iota8 == (toks[slot] & 7)).astype(f32)
            emb_ref[pl.ds(slot, 1), :] = jnp.sum(
                slab_ref[slot] * sel, axis=0, keepdims=True)
        if piece == 0:
            pltpu.make_async_copy(wih0_ref, wih0_ref, wsem.at[0]).wait()
            wih0 = wih0_ref[...]
        else:
            gi_ref[...] = (jnp.dot(emb_ref[...], wih0,
                                   preferred_element_type=f32) + b0i)

    pltpu.make_async_copy(w0_ref, w0_ref, wsem.at[1]).wait()
    pltpu.make_async_copy(wih1_ref, wih1_ref, wsem.at[3]).wait()
    pltpu.make_async_copy(w1_ref, w1_ref, wsem.at[2]).wait()
    pltpu.make_async_copy(wlin_ref, wlin_ref, wsem.at[4]).wait()
    w0 = w0_ref[...]
    w1 = w1_ref[...]
    wih1 = wih1_ref[...]
    wlin = wlin_ref[...]
    blin = blin_ref[...]
    gb0 = ghb_ref[:, 0:H3]
    gb1 = ghb_ref[:, H3:2 * H3]
    b1i = bias1i_ref[...]

    def cell(gi, gh, h):
        rz = jax.nn.sigmoid(gi[:, :2 * H] + gh[:, :2 * H])
        n = jnp.tanh(gi[:, 2 * H:] + rz[:, :H] * gh[:, 2 * H:])
        if h is None:
            return n - rz[:, H:] * n
        return n + rz[:, H:] * (h - n)

    gb0_b = jnp.broadcast_to(gb0, (Bs, H3))
    gb1_b = jnp.broadcast_to(gb1, (Bs, H3))
    h0 = cell(gi_ref[pl.ds(0, Bs), :], gb0_b, None)
    gi1 = [jnp.dot(h0, wih1, preferred_element_type=f32) + b1i]
    h1 = None
    for t in range(1, T):
        gh0 = jnp.dot(h0, w0, preferred_element_type=f32) + gb0
        if t == 1:
            gh1 = gb1_b
        else:
            gh1 = jnp.dot(h1, w1, preferred_element_type=f32) + gb1
        h1 = cell(gi1[t - 1], gh1, h1)
        h1_ref[pl.ds((t - 1) * Bs, Bs), :] = h1
        h0 = cell(gi_ref[pl.ds(t * Bs, Bs), :], gh0, h0)
        gi1.append(jnp.dot(h0, wih1, preferred_element_type=f32) + b1i)
    gh1 = jnp.dot(h1, w1, preferred_element_type=f32) + gb1
    h1 = cell(gi1[T - 1], gh1, h1)
    h1_ref[pl.ds((T - 1) * Bs, Bs), :] = h1

    logits = (jnp.dot(h1_ref[...], wlin,
                      preferred_element_type=f32) + blin)
    rows = [logits[t * Bs:(t + 1) * Bs, :] for t in range(T)]
    m = functools.reduce(jnp.maximum, rows)
    tot = functools.reduce(lambda a, b: a + b,
                           [jnp.exp(r - m) for r in rows])
    lse = m + jnp.log(tot)
    for t in range(T):
        o_ref[:, pl.ds(t, 1), :] = (rows[t] - lse)[:, None, :]


def kernel(tokens, table, wih0, bias0i, whh_blk, gh_bias, wih1, bias1i,
           w_lin, b_lin):
    B, T = tokens.shape
    V, E = table.shape
    C = w_lin.shape[1]
    H3 = wih0.shape[1]
    H = H3 // 3
    G = 1
    Bs = B // G

    def cs(arr):
        nd = arr.ndim
        return pl.BlockSpec(arr.shape, lambda i, tok, _nd=nd: (0,) * _nd)

    hbm = pl.BlockSpec(memory_space=pl.ANY)

    grid_spec = pltpu.PrefetchScalarGridSpec(
        num_scalar_prefetch=1,
        grid=(G,),
        in_specs=[
            hbm,
            hbm,
            hbm,
            hbm,
            hbm,
            cs(bias0i), cs(gh_bias), cs(bias1i), cs(b_lin),
        ],
        out_specs=pl.BlockSpec((Bs, T, C), lambda i, tok: (i, 0, 0)),
        scratch_shapes=[
            pltpu.VMEM((Bs * T, 8, E), jnp.float32),
            pltpu.VMEM((Bs * T, E), jnp.float32),
            pltpu.VMEM((Bs * T, H3), jnp.float32),
            pltpu.VMEM((Bs * T, H), jnp.float32),
            pltpu.VMEM((E, H3), jnp.float32),
            pltpu.VMEM((H, H3), jnp.float32),
            pltpu.VMEM((H, H3), jnp.float32),
            pltpu.VMEM((H, H3), jnp.float32),
            pltpu.VMEM((H, C), jnp.float32),
            pltpu.SemaphoreType.DMA((4,)),
            pltpu.SemaphoreType.DMA((5,)),
        ],
    )
    return pl.pallas_call(
        _slot_rnn_fwd,
        out_shape=jax.ShapeDtypeStruct((B, T, C), jnp.float32),
        grid_spec=grid_spec,
        compiler_params=pltpu.CompilerParams(
            dimension_semantics=("arbitrary",)),
    )(tokens, table, whh_blk, wih0, wih1, w_lin,
      bias0i, gh_bias, bias1i, b_lin)
